# Initial kernel scaffold; baseline (speedup 1.0000x reference)
import jax
import jax.numpy as jnp
from jax.experimental import pallas as pl


def kernel(x, edge_index, edge_attr, batch, eW, eb, c1_W1, c1_b1, c1_g, c1_be, c1_W2, c1_b2, c2_W1, c2_b1, c2_g, c2_be, c2_W2, c2_b2, c3_W1, c3_b1, c3_g, c3_be, c3_W2, c3_b2, fc1_W, fc1_b, fc2_W, fc2_b):
    raise NotImplementedError("write your pallas kernel here")


# speedup vs baseline: 4.9514x; 4.9514x over previous
import functools

import jax
import jax.numpy as jnp
from jax import lax
from jax.experimental import pallas as pl
from jax.experimental.pallas import tpu as pltpu
from jax.experimental.pallas import tpu_sc as plsc

N = 10000
DF = 128
D = 64
G = 128
E = 320000
NPAD = 10112
K = 128
W = 80
NW = 32
EPT = K * W
EPAD = NW * EPT

F32 = jnp.float32
HIGH = lax.Precision.HIGHEST


def _sc_agg(y, src, dst):
    mesh = plsc.VectorSubcoreMesh(core_axis_name="c", subcore_axis_name="s")

    @functools.partial(
        pl.kernel,
        out_type=jax.ShapeDtypeStruct((2, NPAD, D), F32),
        mesh=mesh,
        compiler_params=pltpu.CompilerParams(use_tc_tiling_on_sc=False),
        scratch_types=[
            pltpu.VMEM((4, K), jnp.int32),
            pltpu.VMEM((4, K), jnp.int32),
            pltpu.VMEM((4, K, D), F32),
            pltpu.VMEM_SHARED((NPAD, D), F32),
        ] + [pltpu.SemaphoreType.DMA] * 17,
    )
    def k(y_hbm, s_hbm, d_hbm, out_hbm, sbuf, dbuf, rbuf, acc, *sems):
        sem_si = sems[0:4]
        sem_di = sems[4:8]
        sem_g = sems[8:12]
        sem_sc = sems[12:16]
        sem_w = sems[16]
        c = lax.axis_index("c")
        s = lax.axis_index("s")
        wid = c * 16 + s
        base = wid * EPT
        rps = NPAD // 16
        r0 = s * rps

        pltpu.async_copy(y_hbm.at[pl.ds(r0, rps)],
                         acc.at[pl.ds(r0, rps)], sem_w).wait()
        plsc.subcore_barrier()

        def start_idx(v, j):
            pltpu.async_copy(s_hbm.at[pl.ds(base + v * K, K)],
                             sbuf.at[j], sem_si[j])
            pltpu.async_copy(d_hbm.at[pl.ds(base + v * K, K)],
                             dbuf.at[j], sem_di[j])

        def wait_idx(v, j):
            pltpu.make_async_copy(s_hbm.at[pl.ds(base + v * K, K)],
                                  sbuf.at[j], sem_si[j]).wait()
            pltpu.make_async_copy(d_hbm.at[pl.ds(base + v * K, K)],
                                  dbuf.at[j], sem_di[j]).wait()

        def start_gather(j):
            pltpu.async_copy(y_hbm.at[sbuf.at[j]], rbuf.at[j], sem_g[j])

        def wait_gather(j):
            pltpu.make_async_copy(y_hbm.at[sbuf.at[j]], rbuf.at[j],
                                  sem_g[j]).wait()

        def start_scatter(j):
            pltpu.async_copy(rbuf.at[j], acc.at[dbuf.at[j]], sem_sc[j],
                             add=True)

        def wait_scatter(j):
            pltpu.make_async_copy(rbuf.at[j], acc.at[dbuf.at[j]],
                                  sem_sc[j]).wait()

        def win(v, j, first, last):
            wait_idx(v, j)
            start_gather(j)
            if not (first and j == 0):
                wait_gather((j - 1) % 4)
                start_scatter((j - 1) % 4)
            if not (first and j <= 1):
                wait_scatter((j - 2) % 4)
            if not (last and j >= 2):
                start_idx(v + 2, (j + 2) % 4)

        start_idx(0, 0)
        start_idx(1, 1)
        for j in range(4):
            win(j, j, True, False)

        @pl.loop(4, W - 4, step=4)
        def _(w):
            for j in range(4):
                win(w + j, j, False, False)

        for j in range(4):
            win(W - 4 + j, j, False, True)
        wait_gather(3)
        start_scatter(3)
        wait_scatter(2)
        wait_scatter(3)

        plsc.subcore_barrier()
        pltpu.async_copy(acc.at[pl.ds(r0, rps)],
                         out_hbm.at[c, pl.ds(r0, rps)], sem_w).wait()

    return k(y, src, dst)


def _dot(a, b):
    return jnp.dot(a, b, preferred_element_type=F32, precision=HIGH)


def _tc0_body(x_ref, w_ref, out_ref):
    out_ref[0:N, :] = _dot(x_ref[...], w_ref[...])
    out_ref[N:, :] = jnp.zeros((NPAD - N, D), F32)


def _tc0(x, w):
    return pl.pallas_call(
        _tc0_body,
        out_shape=jax.ShapeDtypeStruct((NPAD, D), F32),
    )(x, w)


def _bn_relu_w2(y_ref, p_ref, b1_ref, g_ref, be_ref, w2_ref, b2_ref):
    s = p_ref[0, 0:N, :] + p_ref[1, 0:N, :] - y_ref[0:N, :] + b1_ref[...]
    mu = jnp.mean(s, axis=0, keepdims=True)
    var = jnp.mean((s - mu) ** 2, axis=0, keepdims=True)
    t = (s - mu) * lax.rsqrt(var + 1e-5) * g_ref[...] + be_ref[...]
    t = jnp.maximum(t, 0.0)
    return jnp.maximum(_dot(t, w2_ref[...]) + b2_ref[...], 0.0)


def _tcmid_body(y_ref, p_ref, b1_ref, g_ref, be_ref, w2_ref, b2_ref,
                w1n_ref, out_ref):
    h = _bn_relu_w2(y_ref, p_ref, b1_ref, g_ref, be_ref, w2_ref, b2_ref)
    out_ref[0:N, :] = _dot(h, w1n_ref[...])
    out_ref[N:, :] = jnp.zeros((NPAD - N, D), F32)


def _tcmid(y, p, b1, g, be, w2, b2, w1n):
    return pl.pallas_call(
        _tcmid_body,
        out_shape=jax.ShapeDtypeStruct((NPAD, D), F32),
    )(y, p, b1, g, be, w2, b2, w1n)


def _tcfin_body(y_ref, p_ref, b1_ref, g_ref, be_ref, w2_ref, b2_ref,
                batch_ref, fc1w_ref, fc1b_ref, fc2w_ref, fc2b_ref, out_ref):
    h = _bn_relu_w2(y_ref, p_ref, b1_ref, g_ref, be_ref, w2_ref, b2_ref)
    seg = lax.broadcasted_iota(jnp.int32, (G, N), 0)
    m = (seg == batch_ref[...]).astype(F32)
    pooled = _dot(m, h)
    o = jnp.maximum(_dot(pooled, fc1w_ref[...]) + fc1b_ref[...], 0.0)
    out_ref[...] = _dot(o, fc2w_ref[...]) + fc2b_ref[...]


def _tcfin(y, p, b1, g, be, w2, b2, batch2d, fc1w, fc1b, fc2w, fc2b):
    return pl.pallas_call(
        _tcfin_body,
        out_shape=jax.ShapeDtypeStruct((G, 1), F32),
    )(y, p, b1, g, be, w2, b2, batch2d, fc1w, fc1b, fc2w, fc2b)


def kernel(x, edge_index, edge_attr, batch, eW, eb,
           c1_W1, c1_b1, c1_g, c1_be, c1_W2, c1_b2,
           c2_W1, c2_b1, c2_g, c2_be, c2_W2, c2_b2,
           c3_W1, c3_b1, c3_g, c3_be, c3_W2, c3_b2,
           fc1_W, fc1_b, fc2_W, fc2_b):
    pad = jnp.full((EPAD - E,), N, jnp.int32)
    src = jnp.concatenate([edge_index[0], pad])
    dst = jnp.concatenate([edge_index[1], pad])
    batch2d = batch.reshape(1, N)

    r2 = lambda v: v.reshape(1, -1)

    y1 = _tc0(x, c1_W1)
    p1 = _sc_agg(y1, src, dst)
    y2 = _tcmid(y1, p1, r2(c1_b1), r2(c1_g), r2(c1_be), c1_W2, r2(c1_b2),
                c2_W1)
    p2 = _sc_agg(y2, src, dst)
    y3 = _tcmid(y2, p2, r2(c2_b1), r2(c2_g), r2(c2_be), c2_W2, r2(c2_b2),
                c3_W1)
    p3 = _sc_agg(y3, src, dst)
    return _tcfin(y3, p3, r2(c3_b1), r2(c3_g), r2(c3_be), c3_W2, r2(c3_b2),
                  batch2d, fc1_W, r2(fc1_b), fc2_W, r2(fc2_b))


# speedup vs baseline: 12.9945x; 2.6244x over previous
import functools

import jax
import jax.numpy as jnp
from jax import lax
from jax.experimental import pallas as pl
from jax.experimental.pallas import tpu as pltpu
from jax.experimental.pallas import tpu_sc as plsc

N = 10000
DF = 128
D = 64
G = 128
E = 320000
NPAD = 10112
K = 128
W = 80
NW = 32
EPT = K * W
EPAD = NW * EPT

F32 = jnp.float32
HIGH = lax.Precision.HIGHEST


def _sc_agg(y, src, dst):
    mesh = plsc.VectorSubcoreMesh(core_axis_name="c", subcore_axis_name="s")

    @functools.partial(
        pl.kernel,
        out_type=jax.ShapeDtypeStruct((2, NPAD, D), F32),
        mesh=mesh,
        compiler_params=pltpu.CompilerParams(use_tc_tiling_on_sc=False),
        scratch_types=[
            pltpu.VMEM((4, K), jnp.int32),
            pltpu.VMEM((4, K), jnp.int32),
            pltpu.VMEM((4, K, D), F32),
            pltpu.VMEM_SHARED((NPAD, D), F32),
        ] + [pltpu.SemaphoreType.DMA] * 17,
    )
    def k(y_hbm, s_hbm, d_hbm, out_hbm, sbuf, dbuf, rbuf, acc, *sems):
        sem_si = sems[0:4]
        sem_di = sems[4:8]
        sem_g = sems[8:12]
        sem_sc = sems[12:16]
        sem_w = sems[16]
        c = lax.axis_index("c")
        s = lax.axis_index("s")
        wid = c * 16 + s
        base = wid * EPT
        rps = NPAD // 16
        r0 = s * rps

        pltpu.async_copy(y_hbm.at[pl.ds(r0, rps)],
                         acc.at[pl.ds(r0, rps)], sem_w).wait()
        plsc.subcore_barrier()

        def start_idx(v, j):
            pltpu.async_copy(s_hbm.at[pl.ds(base + v * K, K)],
                             sbuf.at[j], sem_si[j])
            pltpu.async_copy(d_hbm.at[pl.ds(base + v * K, K)],
                             dbuf.at[j], sem_di[j])

        def wait_idx(v, j):
            pltpu.make_async_copy(s_hbm.at[pl.ds(base + v * K, K)],
                                  sbuf.at[j], sem_si[j]).wait()
            pltpu.make_async_copy(d_hbm.at[pl.ds(base + v * K, K)],
                                  dbuf.at[j], sem_di[j]).wait()

        def start_gather(j):
            pltpu.async_copy(y_hbm.at[sbuf.at[j]], rbuf.at[j], sem_g[j])

        def wait_gather(j):
            pltpu.make_async_copy(y_hbm.at[sbuf.at[j]], rbuf.at[j],
                                  sem_g[j]).wait()

        def start_scatter(j):
            pltpu.async_copy(rbuf.at[j], acc.at[dbuf.at[j]], sem_sc[j],
                             add=True)

        def wait_scatter(j):
            pltpu.make_async_copy(rbuf.at[j], acc.at[dbuf.at[j]],
                                  sem_sc[j]).wait()

        def win(v, j, first, last):
            wait_idx(v, j)
            start_gather(j)
            if not (first and j == 0):
                wait_gather((j - 1) % 4)
                start_scatter((j - 1) % 4)
            if not (first and j <= 1):
                wait_scatter((j - 2) % 4)
            if not (last and j >= 2):
                start_idx(v + 2, (j + 2) % 4)

        start_idx(0, 0)
        start_idx(1, 1)
        for j in range(4):
            win(j, j, True, False)

        @pl.loop(4, W - 4, step=4)
        def _(w):
            for j in range(4):
                win(w + j, j, False, False)

        for j in range(4):
            win(W - 4 + j, j, False, True)
        wait_gather(3)
        start_scatter(3)
        wait_scatter(2)
        wait_scatter(3)

        plsc.subcore_barrier()
        pltpu.async_copy(acc.at[pl.ds(r0, rps)],
                         out_hbm.at[c, pl.ds(r0, rps)], sem_w).wait()

    return k(y, src, dst)


def _dot(a, b):
    return jnp.dot(a, b, preferred_element_type=F32, precision=HIGH)


def _tc0_body(x_ref, w_ref, out_ref):
    out_ref[0:N, :] = _dot(x_ref[...], w_ref[...])
    out_ref[N:, :] = jnp.zeros((NPAD - N, D), F32)


def _tc0(x, w):
    return pl.pallas_call(
        _tc0_body,
        out_shape=jax.ShapeDtypeStruct((NPAD, D), F32),
    )(x, w)


def _bn_relu_w2(y_ref, p_ref, b1_ref, g_ref, be_ref, w2_ref, b2_ref):
    s = p_ref[0, 0:N, :] + p_ref[1, 0:N, :] - y_ref[0:N, :] + b1_ref[...]
    mu = jnp.mean(s, axis=0, keepdims=True)
    var = jnp.mean((s - mu) ** 2, axis=0, keepdims=True)
    t = (s - mu) * lax.rsqrt(var + 1e-5) * g_ref[...] + be_ref[...]
    t = jnp.maximum(t, 0.0)
    return jnp.maximum(_dot(t, w2_ref[...]) + b2_ref[...], 0.0)


def _tcmid_body(y_ref, p_ref, b1_ref, g_ref, be_ref, w2_ref, b2_ref,
                w1n_ref, out_ref):
    h = _bn_relu_w2(y_ref, p_ref, b1_ref, g_ref, be_ref, w2_ref, b2_ref)
    out_ref[0:N, :] = _dot(h, w1n_ref[...])
    out_ref[N:, :] = jnp.zeros((NPAD - N, D), F32)


def _tcmid(y, p, b1, g, be, w2, b2, w1n):
    return pl.pallas_call(
        _tcmid_body,
        out_shape=jax.ShapeDtypeStruct((NPAD, D), F32),
    )(y, p, b1, g, be, w2, b2, w1n)


def _tcfin_body(y_ref, p_ref, b1_ref, g_ref, be_ref, w2_ref, b2_ref,
                batch_ref, fc1w_ref, fc1b_ref, fc2w_ref, fc2b_ref, out_ref):
    h = _bn_relu_w2(y_ref, p_ref, b1_ref, g_ref, be_ref, w2_ref, b2_ref)
    seg = lax.broadcasted_iota(jnp.int32, (G, N), 0)
    m = (seg == batch_ref[...]).astype(F32)
    pooled = _dot(m, h)
    o = jnp.maximum(_dot(pooled, fc1w_ref[...]) + fc1b_ref[...], 0.0)
    out_ref[...] = _dot(o, fc2w_ref[...]) + fc2b_ref[...]


def _tcfin(y, p, b1, g, be, w2, b2, batch2d, fc1w, fc1b, fc2w, fc2b):
    return pl.pallas_call(
        _tcfin_body,
        out_shape=jax.ShapeDtypeStruct((G, 1), F32),
    )(y, p, b1, g, be, w2, b2, batch2d, fc1w, fc1b, fc2w, fc2b)


def kernel(x, edge_index, edge_attr, batch, eW, eb,
           c1_W1, c1_b1, c1_g, c1_be, c1_W2, c1_b2,
           c2_W1, c2_b1, c2_g, c2_be, c2_W2, c2_b2,
           c3_W1, c3_b1, c3_g, c3_be, c3_W2, c3_b2,
           fc1_W, fc1_b, fc2_W, fc2_b):
    pad = N + jnp.arange(EPAD - E, dtype=jnp.int32) % (NPAD - N)
    src = jnp.concatenate([edge_index[0], pad])
    dst = jnp.concatenate([edge_index[1], pad])
    batch2d = batch.reshape(1, N)

    r2 = lambda v: v.reshape(1, -1)

    y1 = _tc0(x, c1_W1)
    p1 = _sc_agg(y1, src, dst)
    y2 = _tcmid(y1, p1, r2(c1_b1), r2(c1_g), r2(c1_be), c1_W2, r2(c1_b2),
                c2_W1)
    p2 = _sc_agg(y2, src, dst)
    y3 = _tcmid(y2, p2, r2(c2_b1), r2(c2_g), r2(c2_be), c2_W2, r2(c2_b2),
                c3_W1)
    p3 = _sc_agg(y3, src, dst)
    return _tcfin(y3, p3, r2(c3_b1), r2(c3_g), r2(c3_be), c3_W2, r2(c3_b2),
                  batch2d, fc1_W, r2(fc1_b), fc2_W, r2(fc2_b))


# speedup vs baseline: 14.9030x; 1.1469x over previous
import functools

import jax
import jax.numpy as jnp
from jax import lax
from jax.experimental import pallas as pl
from jax.experimental.pallas import tpu as pltpu
from jax.experimental.pallas import tpu_sc as plsc

N = 10000
DF = 128
D = 64
G = 128
E = 320000
NPAD = 10112
K = 128
W = 80
NW = 32
EPT = K * W
EPAD = NW * EPT

F32 = jnp.float32
HIGH = lax.Precision.DEFAULT


def _sc_agg(y, src, dst):
    mesh = plsc.VectorSubcoreMesh(core_axis_name="c", subcore_axis_name="s")

    @functools.partial(
        pl.kernel,
        out_type=jax.ShapeDtypeStruct((2, NPAD, D), F32),
        mesh=mesh,
        compiler_params=pltpu.CompilerParams(use_tc_tiling_on_sc=False),
        scratch_types=[
            pltpu.VMEM((4, K), jnp.int32),
            pltpu.VMEM((4, K), jnp.int32),
            pltpu.VMEM((4, K, D), F32),
            pltpu.VMEM_SHARED((NPAD, D), F32),
        ] + [pltpu.SemaphoreType.DMA] * 17,
    )
    def k(y_hbm, s_hbm, d_hbm, out_hbm, sbuf, dbuf, rbuf, acc, *sems):
        sem_si = sems[0:4]
        sem_di = sems[4:8]
        sem_g = sems[8:12]
        sem_sc = sems[12:16]
        sem_w = sems[16]
        c = lax.axis_index("c")
        s = lax.axis_index("s")
        wid = c * 16 + s
        base = wid * EPT
        rps = NPAD // 16
        r0 = s * rps

        pltpu.async_copy(y_hbm.at[pl.ds(r0, rps)],
                         acc.at[pl.ds(r0, rps)], sem_w).wait()
        plsc.subcore_barrier()

        def start_idx(v, j):
            pltpu.async_copy(s_hbm.at[pl.ds(base + v * K, K)],
                             sbuf.at[j], sem_si[j])
            pltpu.async_copy(d_hbm.at[pl.ds(base + v * K, K)],
                             dbuf.at[j], sem_di[j])

        def wait_idx(v, j):
            pltpu.make_async_copy(s_hbm.at[pl.ds(base + v * K, K)],
                                  sbuf.at[j], sem_si[j]).wait()
            pltpu.make_async_copy(d_hbm.at[pl.ds(base + v * K, K)],
                                  dbuf.at[j], sem_di[j]).wait()

        def start_gather(j):
            pltpu.async_copy(y_hbm.at[sbuf.at[j]], rbuf.at[j], sem_g[j])

        def wait_gather(j):
            pltpu.make_async_copy(y_hbm.at[sbuf.at[j]], rbuf.at[j],
                                  sem_g[j]).wait()

        def start_scatter(j):
            pltpu.async_copy(rbuf.at[j], acc.at[dbuf.at[j]], sem_sc[j],
                             add=True)

        def wait_scatter(j):
            pltpu.make_async_copy(rbuf.at[j], acc.at[dbuf.at[j]],
                                  sem_sc[j]).wait()

        def win(v, j, first, last):
            wait_idx(v, j)
            start_gather(j)
            if not (first and j == 0):
                wait_gather((j - 1) % 4)
                start_scatter((j - 1) % 4)
            if not (first and j <= 1):
                wait_scatter((j - 2) % 4)
            if not (last and j >= 2):
                start_idx(v + 2, (j + 2) % 4)

        start_idx(0, 0)
        start_idx(1, 1)
        for j in range(4):
            win(j, j, True, False)

        @pl.loop(4, W - 4, step=4)
        def _(w):
            for j in range(4):
                win(w + j, j, False, False)

        for j in range(4):
            win(W - 4 + j, j, False, True)
        wait_gather(3)
        start_scatter(3)
        wait_scatter(2)
        wait_scatter(3)

        plsc.subcore_barrier()
        pltpu.async_copy(acc.at[pl.ds(r0, rps)],
                         out_hbm.at[c, pl.ds(r0, rps)], sem_w).wait()

    return k(y, src, dst)


def _dot(a, b):
    return jnp.dot(a, b, preferred_element_type=F32, precision=HIGH)


def _tc0_body(x_ref, w_ref, out_ref):
    out_ref[0:N, :] = _dot(x_ref[...], w_ref[...])
    out_ref[N:, :] = jnp.zeros((NPAD - N, D), F32)


def _tc0(x, w):
    return pl.pallas_call(
        _tc0_body,
        out_shape=jax.ShapeDtypeStruct((NPAD, D), F32),
    )(x, w)


def _bn_relu_w2(y_ref, p_ref, b1_ref, g_ref, be_ref, w2_ref, b2_ref):
    s = p_ref[0, 0:N, :] + p_ref[1, 0:N, :] - y_ref[0:N, :] + b1_ref[...]
    mu = jnp.mean(s, axis=0, keepdims=True)
    var = jnp.mean((s - mu) ** 2, axis=0, keepdims=True)
    t = (s - mu) * lax.rsqrt(var + 1e-5) * g_ref[...] + be_ref[...]
    t = jnp.maximum(t, 0.0)
    return jnp.maximum(_dot(t, w2_ref[...]) + b2_ref[...], 0.0)


def _tcmid_body(y_ref, p_ref, b1_ref, g_ref, be_ref, w2_ref, b2_ref,
                w1n_ref, out_ref):
    h = _bn_relu_w2(y_ref, p_ref, b1_ref, g_ref, be_ref, w2_ref, b2_ref)
    out_ref[0:N, :] = _dot(h, w1n_ref[...])
    out_ref[N:, :] = jnp.zeros((NPAD - N, D), F32)


def _tcmid(y, p, b1, g, be, w2, b2, w1n):
    return pl.pallas_call(
        _tcmid_body,
        out_shape=jax.ShapeDtypeStruct((NPAD, D), F32),
    )(y, p, b1, g, be, w2, b2, w1n)


def _tcfin_body(y_ref, p_ref, b1_ref, g_ref, be_ref, w2_ref, b2_ref,
                batch_ref, fc1w_ref, fc1b_ref, fc2w_ref, fc2b_ref, out_ref):
    h = _bn_relu_w2(y_ref, p_ref, b1_ref, g_ref, be_ref, w2_ref, b2_ref)
    seg = lax.broadcasted_iota(jnp.int32, (G, N), 0)
    m = (seg == batch_ref[...]).astype(F32)
    pooled = jnp.dot(m, h, preferred_element_type=F32,
                     precision=lax.Precision.HIGHEST)
    o = jnp.maximum(_dot(pooled, fc1w_ref[...]) + fc1b_ref[...], 0.0)
    out_ref[...] = _dot(o, fc2w_ref[...]) + fc2b_ref[...]


def _tcfin(y, p, b1, g, be, w2, b2, batch2d, fc1w, fc1b, fc2w, fc2b):
    return pl.pallas_call(
        _tcfin_body,
        out_shape=jax.ShapeDtypeStruct((G, 1), F32),
    )(y, p, b1, g, be, w2, b2, batch2d, fc1w, fc1b, fc2w, fc2b)


def kernel(x, edge_index, edge_attr, batch, eW, eb,
           c1_W1, c1_b1, c1_g, c1_be, c1_W2, c1_b2,
           c2_W1, c2_b1, c2_g, c2_be, c2_W2, c2_b2,
           c3_W1, c3_b1, c3_g, c3_be, c3_W2, c3_b2,
           fc1_W, fc1_b, fc2_W, fc2_b):
    pad = N + jnp.arange(EPAD - E, dtype=jnp.int32) % (NPAD - N)
    src = jnp.concatenate([edge_index[0], pad])
    dst = jnp.concatenate([edge_index[1], pad])
    batch2d = batch.reshape(1, N)

    r2 = lambda v: v.reshape(1, -1)

    y1 = _tc0(x, c1_W1)
    p1 = _sc_agg(y1, src, dst)
    y2 = _tcmid(y1, p1, r2(c1_b1), r2(c1_g), r2(c1_be), c1_W2, r2(c1_b2),
                c2_W1)
    p2 = _sc_agg(y2, src, dst)
    y3 = _tcmid(y2, p2, r2(c2_b1), r2(c2_g), r2(c2_be), c2_W2, r2(c2_b2),
                c3_W1)
    p3 = _sc_agg(y3, src, dst)
    return _tcfin(y3, p3, r2(c3_b1), r2(c3_g), r2(c3_be), c3_W2, r2(c3_b2),
                  batch2d, fc1_W, r2(fc1_b), fc2_W, r2(fc2_b))
